# Initial kernel scaffold; baseline (speedup 1.0000x reference)
import jax
import jax.numpy as jnp
from jax.experimental import pallas as pl


def kernel(x, edge_index, node_graph_ids, W_lift, b_lift, W1, b1, W2, b2, W3, b3, Wr, br):
    raise NotImplementedError("write your pallas kernel here")


# speedup vs baseline: 8.5424x; 8.5424x over previous
import functools

import jax
import jax.numpy as jnp
from jax import lax
from jax.experimental import pallas as pl
from jax.experimental.pallas import tpu as pltpu
from jax.experimental.pallas import tpu_sc as plsc

N_NODES = 50000
HIDDEN = 32
N_EDGES = 1600000
NUM_GRAPHS = 64

NC = 2
NS = 16
NW = NC * NS

NPAD = 50048
ZROWS = NPAD // NS
SCRAP = N_NODES

EB = 128
EW = 50048
TCHUNKS = EW // EB
EPAD = NW * EW

RBLK = 2176
NBLK = NPAD // RBLK


def _sc_layer_body(h_hbm, src_hbm, dst_hbm, zeros_hbm, out_hbm,
                   idx_s, idx_d, rows, shared, sem):
    c = lax.axis_index("c")
    s = lax.axis_index("s")
    wid = c * NS + s
    row0 = s * ZROWS

    pltpu.sync_copy(zeros_hbm, shared.at[pl.ds(row0, ZROWS)])
    plsc.subcore_barrier()

    ebase = wid * EW

    @pl.loop(0, TCHUNKS)
    def _(g):
        off = ebase + g * EB
        pltpu.sync_copy(src_hbm.at[pl.ds(off, EB)], idx_s)
        pltpu.async_copy(h_hbm.at[idx_s], rows, sem).wait()
        pltpu.sync_copy(dst_hbm.at[pl.ds(off, EB)], idx_d)
        pltpu.sync_copy(rows, shared.at[idx_d], add=True)

    plsc.subcore_barrier()
    pltpu.sync_copy(shared.at[pl.ds(row0, ZROWS)],
                    out_hbm.at[c].at[pl.ds(row0, ZROWS)])


def _sc_segment_sum(h, src, dst, zeros):
    mesh = plsc.VectorSubcoreMesh(core_axis_name="c", subcore_axis_name="s",
                                  num_cores=NC, num_subcores=NS)
    return pl.kernel(
        _sc_layer_body,
        out_type=jax.ShapeDtypeStruct((NC, NPAD, HIDDEN), jnp.float32),
        mesh=mesh,
        scratch_types=[
            pltpu.VMEM((EB,), jnp.int32),
            pltpu.VMEM((EB,), jnp.int32),
            pltpu.VMEM((EB, HIDDEN), jnp.float32),
            pltpu.VMEM_SHARED((NPAD, HIDDEN), jnp.float32),
            pltpu.SemaphoreType.DMA,
        ],
        compiler_params=pltpu.CompilerParams(use_tc_tiling_on_sc=False),
    )(h, src, dst, zeros)


def _lift_body(x_ref, w_ref, b_ref, o_ref):
    o_ref[...] = (
        jnp.dot(x_ref[...], w_ref[...], preferred_element_type=jnp.float32)
        + b_ref[...]
    )


def _tc_lift(x_pad, w_pad, b):
    return pl.pallas_call(
        _lift_body,
        grid=(NBLK,),
        in_specs=[
            pl.BlockSpec((RBLK, 128), lambda i: (i, 0)),
            pl.BlockSpec((128, HIDDEN), lambda i: (0, 0)),
            pl.BlockSpec((1, HIDDEN), lambda i: (0, 0)),
        ],
        out_specs=pl.BlockSpec((RBLK, HIDDEN), lambda i: (i, 0)),
        out_shape=jax.ShapeDtypeStruct((NPAD, HIDDEN), jnp.float32),
    )(x_pad, w_pad, b.reshape(1, HIDDEN))


def _layer_body(p_ref, w_ref, b_ref, o_ref):
    agg = p_ref[0] + p_ref[1]
    h = jnp.dot(agg, w_ref[...], preferred_element_type=jnp.float32) + b_ref[...]
    o_ref[...] = jnp.maximum(h, 0.0)


def _tc_layer(parts, w, b):
    return pl.pallas_call(
        _layer_body,
        grid=(NBLK,),
        in_specs=[
            pl.BlockSpec((NC, RBLK, HIDDEN), lambda i: (0, i, 0)),
            pl.BlockSpec((HIDDEN, HIDDEN), lambda i: (0, 0)),
            pl.BlockSpec((1, HIDDEN), lambda i: (0, 0)),
        ],
        out_specs=pl.BlockSpec((RBLK, HIDDEN), lambda i: (i, 0)),
        out_shape=jax.ShapeDtypeStruct((NPAD, HIDDEN), jnp.float32),
    )(parts, w, b.reshape(1, HIDDEN))


def _readout_body(h_ref, ids_ref, wr_ref, br_ref, o_ref, seg, cnt):
    i = pl.program_id(0)

    @pl.when(i == 0)
    def _():
        seg[...] = jnp.zeros_like(seg)
        cnt[...] = jnp.zeros_like(cnt)

    ids = ids_ref[0, 0, :]
    oh_t = (ids[None, :]
            == lax.broadcasted_iota(jnp.int32, (NUM_GRAPHS, RBLK), 0)
            ).astype(jnp.float32)
    seg[...] += jax.lax.dot(oh_t, h_ref[...],
                            preferred_element_type=jnp.float32)
    cnt[...] += jnp.sum(oh_t, axis=1, keepdims=True)

    @pl.when(i == NBLK - 1)
    def _():
        o_ref[...] = (
            jax.lax.dot(seg[...], wr_ref[...],
                        preferred_element_type=jnp.float32)
            + cnt[...] * br_ref[...]
        )


def _tc_readout(h, ids3, wr, br):
    return pl.pallas_call(
        _readout_body,
        grid=(NBLK,),
        in_specs=[
            pl.BlockSpec((RBLK, HIDDEN), lambda i: (i, 0)),
            pl.BlockSpec((1, 1, RBLK), lambda i: (i, 0, 0)),
            pl.BlockSpec((HIDDEN, 2), lambda i: (0, 0)),
            pl.BlockSpec((1, 2), lambda i: (0, 0)),
        ],
        out_specs=pl.BlockSpec((NUM_GRAPHS, 2), lambda i: (0, 0)),
        out_shape=jax.ShapeDtypeStruct((NUM_GRAPHS, 2), jnp.float32),
        scratch_shapes=[
            pltpu.VMEM((NUM_GRAPHS, HIDDEN), jnp.float32),
            pltpu.VMEM((NUM_GRAPHS, 1), jnp.float32),
        ],
    )(h, ids3, wr, br.reshape(1, 2))


@jax.jit
def kernel(x, edge_index, node_graph_ids, W_lift, b_lift, W1, b1, W2, b2,
           W3, b3, Wr, br):
    x_pad = jnp.pad(x, ((0, NPAD - N_NODES), (0, 128 - x.shape[1])))
    wl_pad = jnp.pad(W_lift, ((0, 128 - W_lift.shape[0]), (0, 0)))
    src = jnp.concatenate(
        [edge_index[0], jnp.zeros((EPAD - N_EDGES,), jnp.int32)])
    dst = jnp.concatenate(
        [edge_index[1], jnp.full((EPAD - N_EDGES,), SCRAP, jnp.int32)])
    ids3 = jnp.pad(node_graph_ids, (0, NPAD - N_NODES),
                   constant_values=NUM_GRAPHS).reshape(NBLK, 1, RBLK)
    zeros = jnp.zeros((ZROWS, HIDDEN), jnp.float32)

    h = _tc_lift(x_pad, wl_pad, b_lift)

    for (w, b) in ((W1, b1), (W2, b2), (W3, b3)):
        parts = _sc_segment_sum(h, src, dst, zeros)
        h = _tc_layer(parts, w, b)

    return _tc_readout(h, ids3, Wr, br)


# speedup vs baseline: 19.1082x; 2.2369x over previous
import functools

import jax
import jax.numpy as jnp
from jax import lax
from jax.experimental import pallas as pl
from jax.experimental.pallas import tpu as pltpu
from jax.experimental.pallas import tpu_sc as plsc

N_NODES = 50000
HIDDEN = 32
N_EDGES = 1600000
NUM_GRAPHS = 64

NC = 2
NS = 16
NW = NC * NS

NPAD = 50048
ZROWS = NPAD // NS
SCRAP = N_NODES

EB = 128
KROWS = 7
EW = 50176
MCHUNKS = EW // (KROWS * EB)
EPAD = NW * EW
EROWS = EPAD // EB

RBLK = 2176
NBLK = NPAD // RBLK


def _sc_layer_body(h_hbm, src_hbm, dst_hbm, zeros_hbm, out_hbm,
                   idx_s, idx_d, rows, shared, gsem, ssem):
    c = lax.axis_index("c")
    s = lax.axis_index("s")
    wid = c * NS + s
    row0 = s * ZROWS

    pltpu.sync_copy(zeros_hbm, shared.at[pl.ds(row0, ZROWS)])
    plsc.subcore_barrier()

    ebase = wid * (EW // EB)

    @pl.loop(0, MCHUNKS)
    def _(g):
        off = ebase + g * KROWS
        pltpu.sync_copy(src_hbm.at[pl.ds(off, KROWS)], idx_s)
        pltpu.sync_copy(dst_hbm.at[pl.ds(off, KROWS)], idx_d)
        gd = [pltpu.async_copy(h_hbm.at[idx_s.at[j]], rows.at[j], gsem)
              for j in range(KROWS)]
        sd = []
        for j in range(KROWS):
            gd[j].wait()
            sd.append(pltpu.async_copy(rows.at[j], shared.at[idx_d.at[j]],
                                       ssem, add=True))
        for d in sd:
            d.wait()

    plsc.subcore_barrier()
    pltpu.sync_copy(shared.at[pl.ds(row0, ZROWS)],
                    out_hbm.at[c].at[pl.ds(row0, ZROWS)])


def _sc_segment_sum(h, src, dst, zeros):
    mesh = plsc.VectorSubcoreMesh(core_axis_name="c", subcore_axis_name="s",
                                  num_cores=NC, num_subcores=NS)
    return pl.kernel(
        _sc_layer_body,
        out_type=jax.ShapeDtypeStruct((NC, NPAD, HIDDEN), jnp.float32),
        mesh=mesh,
        scratch_types=[
            pltpu.VMEM((KROWS, EB), jnp.int32),
            pltpu.VMEM((KROWS, EB), jnp.int32),
            pltpu.VMEM((KROWS, EB, HIDDEN), jnp.float32),
            pltpu.VMEM_SHARED((NPAD, HIDDEN), jnp.float32),
            pltpu.SemaphoreType.DMA,
            pltpu.SemaphoreType.DMA,
        ],
        compiler_params=pltpu.CompilerParams(use_tc_tiling_on_sc=False),
    )(h, src, dst, zeros)


def _lift_body(x_ref, w_ref, b_ref, o_ref):
    o_ref[...] = (
        jnp.dot(x_ref[...], w_ref[...], preferred_element_type=jnp.float32)
        + b_ref[...]
    )


def _tc_lift(x_pad, w_pad, b):
    return pl.pallas_call(
        _lift_body,
        grid=(NBLK,),
        in_specs=[
            pl.BlockSpec((RBLK, 128), lambda i: (i, 0)),
            pl.BlockSpec((128, HIDDEN), lambda i: (0, 0)),
            pl.BlockSpec((1, HIDDEN), lambda i: (0, 0)),
        ],
        out_specs=pl.BlockSpec((RBLK, HIDDEN), lambda i: (i, 0)),
        out_shape=jax.ShapeDtypeStruct((NPAD, HIDDEN), jnp.float32),
    )(x_pad, w_pad, b.reshape(1, HIDDEN))


def _layer_body(p_ref, w_ref, b_ref, o_ref):
    agg = p_ref[0] + p_ref[1]
    h = jnp.dot(agg, w_ref[...], preferred_element_type=jnp.float32) + b_ref[...]
    o_ref[...] = jnp.maximum(h, 0.0)


def _tc_layer(parts, w, b):
    return pl.pallas_call(
        _layer_body,
        grid=(NBLK,),
        in_specs=[
            pl.BlockSpec((NC, RBLK, HIDDEN), lambda i: (0, i, 0)),
            pl.BlockSpec((HIDDEN, HIDDEN), lambda i: (0, 0)),
            pl.BlockSpec((1, HIDDEN), lambda i: (0, 0)),
        ],
        out_specs=pl.BlockSpec((RBLK, HIDDEN), lambda i: (i, 0)),
        out_shape=jax.ShapeDtypeStruct((NPAD, HIDDEN), jnp.float32),
    )(parts, w, b.reshape(1, HIDDEN))


def _readout_body(h_ref, ids_ref, wr_ref, br_ref, o_ref, seg, cnt):
    i = pl.program_id(0)

    @pl.when(i == 0)
    def _():
        seg[...] = jnp.zeros_like(seg)
        cnt[...] = jnp.zeros_like(cnt)

    ids = ids_ref[0, 0, :]
    oh_t = (ids[None, :]
            == lax.broadcasted_iota(jnp.int32, (NUM_GRAPHS, RBLK), 0)
            ).astype(jnp.float32)
    seg[...] += jax.lax.dot(oh_t, h_ref[...],
                            preferred_element_type=jnp.float32)
    cnt[...] += jnp.sum(oh_t, axis=1, keepdims=True)

    @pl.when(i == NBLK - 1)
    def _():
        o_ref[...] = (
            jax.lax.dot(seg[...], wr_ref[...],
                        preferred_element_type=jnp.float32)
            + cnt[...] * br_ref[...]
        )


def _tc_readout(h, ids3, wr, br):
    return pl.pallas_call(
        _readout_body,
        grid=(NBLK,),
        in_specs=[
            pl.BlockSpec((RBLK, HIDDEN), lambda i: (i, 0)),
            pl.BlockSpec((1, 1, RBLK), lambda i: (i, 0, 0)),
            pl.BlockSpec((HIDDEN, 2), lambda i: (0, 0)),
            pl.BlockSpec((1, 2), lambda i: (0, 0)),
        ],
        out_specs=pl.BlockSpec((NUM_GRAPHS, 2), lambda i: (0, 0)),
        out_shape=jax.ShapeDtypeStruct((NUM_GRAPHS, 2), jnp.float32),
        scratch_shapes=[
            pltpu.VMEM((NUM_GRAPHS, HIDDEN), jnp.float32),
            pltpu.VMEM((NUM_GRAPHS, 1), jnp.float32),
        ],
    )(h, ids3, wr, br.reshape(1, 2))


@jax.jit
def kernel(x, edge_index, node_graph_ids, W_lift, b_lift, W1, b1, W2, b2,
           W3, b3, Wr, br):
    x_pad = jnp.pad(x, ((0, NPAD - N_NODES), (0, 128 - x.shape[1])))
    wl_pad = jnp.pad(W_lift, ((0, 128 - W_lift.shape[0]), (0, 0)))
    src = jnp.concatenate(
        [edge_index[0], jnp.zeros((EPAD - N_EDGES,), jnp.int32)]
    ).reshape(EROWS, EB)
    dst = jnp.concatenate(
        [edge_index[1], jnp.full((EPAD - N_EDGES,), SCRAP, jnp.int32)]
    ).reshape(EROWS, EB)
    ids3 = jnp.pad(node_graph_ids, (0, NPAD - N_NODES),
                   constant_values=NUM_GRAPHS).reshape(NBLK, 1, RBLK)
    zeros = jnp.zeros((ZROWS, HIDDEN), jnp.float32)

    h = _tc_lift(x_pad, wl_pad, b_lift)

    for (w, b) in ((W1, b1), (W2, b2), (W3, b3)):
        parts = _sc_segment_sum(h, src, dst, zeros)
        h = _tc_layer(parts, w, b)

    return _tc_readout(h, ids3, Wr, br)
